# baseline (speedup 1.0000x reference)
import functools

import jax
import jax.numpy as jnp
from jax import lax
from jax.experimental import pallas as pl
from jax.experimental.pallas import tpu as pltpu
from jax.experimental.pallas import tpu_sc as plsc

_K = 512
_D = 128
_B = 16
_L = 4096
_LBLK = 2048
_RC = 8

_NW = 32
_CHUNK = 256


def _idx_body(z_ref, w_ref, idx_ref, loss_ref):
    b = pl.program_id(0)
    l = pl.program_id(1)
    zb = z_ref[0]
    w = w_ref[...]
    fz2 = jnp.sum(zb * zb, axis=0, keepdims=True)
    w2 = jnp.sum(w * w, axis=1, keepdims=True)
    dot2 = jax.lax.dot_general(
        w * 2.0, zb, (((1,), (0,)), ((), ())),
        preferred_element_type=jnp.float32)

    accv = (fz2 + w2[0:_RC]) - dot2[0:_RC]
    acci = jnp.zeros((_RC, _LBLK), jnp.int32)
    for i in range(1, _K // _RC):
        dch = (fz2 + w2[i * _RC:(i + 1) * _RC]) - dot2[i * _RC:(i + 1) * _RC]
        lt = dch < accv
        accv = jnp.where(lt, dch, accv)
        acci = jnp.where(lt, i, acci)

    minv = jnp.min(accv, axis=0, keepdims=True)
    siota = jax.lax.broadcasted_iota(jnp.int32, (_RC, _LBLK), 0)
    fidx = acci * _RC + siota
    idx = jnp.min(jnp.where(accv == minv, fidx, _K),
                  axis=0, keepdims=True)
    idx_ref[0] = idx

    @pl.when((b == 0) & (l == 0))
    def _init():
        loss_ref[...] = jnp.zeros((1, 1), jnp.float32)

    loss_ref[...] += jnp.sum(minv, axis=1, keepdims=True)


def _tc_indices(z, W):
    grid = (_B, _L // _LBLK)
    idx, loss = pl.pallas_call(
        _idx_body,
        grid=grid,
        in_specs=[
            pl.BlockSpec((1, _D, _LBLK), lambda b, l: (b, 0, l)),
            pl.BlockSpec((_K, _D), lambda b, l: (0, 0)),
        ],
        out_specs=[
            pl.BlockSpec((1, 1, _LBLK), lambda b, l: (b, 0, l)),
            pl.BlockSpec((1, 1), lambda b, l: (0, 0)),
        ],
        out_shape=[
            jax.ShapeDtypeStruct((_B, 1, _L), jnp.int32),
            jax.ShapeDtypeStruct((1, 1), jnp.float32),
        ],
    )(z, W)
    return idx, loss


def _make_sc_gather():
    mesh = plsc.VectorSubcoreMesh(core_axis_name="c", subcore_axis_name="s",
                                  num_cores=2, num_subcores=16)

    @functools.partial(
        pl.kernel, mesh=mesh,
        out_type=jax.ShapeDtypeStruct((_B, _D, _L), jnp.float32),
        scratch_types=[
            pltpu.VMEM((_D * _K,), jnp.float32),
            pltpu.VMEM((_CHUNK,), jnp.int32),
            pltpu.VMEM((_D, _CHUNK), jnp.float32),
        ],
        compiler_params=pltpu.CompilerParams(use_tc_tiling_on_sc=False,
                                             needs_layout_passes=False),
    )
    def sc_gather(wt_hbm, idx_hbm, out_hbm, wt_v, idx_v, stage_v):
        wid = lax.axis_index("s") * 2 + lax.axis_index("c")
        pltpu.sync_copy(wt_hbm, wt_v)
        b = wid // 2
        lhalf = (wid % 2) * (_L // 2)

        def sub_body(sub, _):
            lbase = lhalf + sub * _CHUNK
            pltpu.sync_copy(idx_hbm.at[b, 0, pl.ds(lbase, _CHUNK)], idx_v)

            def j_body(j, _):
                iv = idx_v[pl.ds(j * 16, 16)]
                for c in range(_D):
                    row = plsc.load_gather(wt_v, [iv + (c * _K)])
                    stage_v[c, pl.ds(j * 16, 16)] = row
                return 0

            lax.fori_loop(0, _CHUNK // 16, j_body, 0)
            pltpu.sync_copy(stage_v, out_hbm.at[b, :, pl.ds(lbase, _CHUNK)])
            return 0

        lax.fori_loop(0, (_L // 2) // _CHUNK, sub_body, 0)

    return sc_gather


_sc_gather = _make_sc_gather()


@jax.jit
def kernel(z, W):
    idx, loss = _tc_indices(z, W)
    wt = W.T.reshape(-1)
    out = _sc_gather(wt, idx)
    scale = 1.25 / (_B * _L * _D)
    return out, (loss[0, 0] * scale).astype(jnp.float32)


# speedup vs baseline: 3.8674x; 3.8674x over previous
import jax
import jax.numpy as jnp
from jax.experimental import pallas as pl

_K = 512
_D = 128
_B = 16
_L = 4096
_LBLK = 4096
_RC = 8


def _vq_body(z_ref, w_ref, out_ref, loss_ref):
    b = pl.program_id(0)
    l = pl.program_id(1)

    zb = z_ref[0]
    w = w_ref[...]

    fz2 = jnp.sum(zb * zb, axis=0, keepdims=True)
    w2 = jnp.sum(w * w, axis=1, keepdims=True)
    dot2 = jax.lax.dot_general(
        w * 2.0, zb, (((1,), (0,)), ((), ())),
        preferred_element_type=jnp.float32)

    accv = (fz2 + w2[0:_RC]) - dot2[0:_RC]
    acci = jnp.zeros((_RC, _LBLK), jnp.int32)
    for i in range(1, _K // _RC):
        dch = (fz2 + w2[i * _RC:(i + 1) * _RC]) - dot2[i * _RC:(i + 1) * _RC]
        lt = dch < accv
        accv = jnp.where(lt, dch, accv)
        acci = jnp.where(lt, i, acci)

    minv = jnp.min(accv, axis=0, keepdims=True)
    siota = jax.lax.broadcasted_iota(jnp.int32, (_RC, _LBLK), 0)
    fidx = acci * _RC + siota
    idx = jnp.min(jnp.where(accv == minv, fidx, _K),
                  axis=0, keepdims=True)
    kiota = jax.lax.broadcasted_iota(jnp.int32, (_K, _LBLK), 0)
    onehot = (kiota == idx).astype(jnp.float32)

    q = jax.lax.dot_general(
        w, onehot, (((0,), (0,)), ((), ())),
        preferred_element_type=jnp.float32)
    out_ref[0] = q

    @pl.when((b == 0) & (l == 0))
    def _init():
        loss_ref[...] = jnp.zeros((1, 1), jnp.float32)

    loss_ref[...] += jnp.sum(minv, axis=1, keepdims=True)


@jax.jit
def kernel(z, W):
    grid = (_B, _L // _LBLK)
    out, loss = pl.pallas_call(
        _vq_body,
        grid=grid,
        in_specs=[
            pl.BlockSpec((1, _D, _LBLK), lambda b, l: (b, 0, l)),
            pl.BlockSpec((_K, _D), lambda b, l: (0, 0)),
        ],
        out_specs=[
            pl.BlockSpec((1, _D, _LBLK), lambda b, l: (b, 0, l)),
            pl.BlockSpec((1, 1), lambda b, l: (0, 0)),
        ],
        out_shape=[
            jax.ShapeDtypeStruct(z.shape, jnp.float32),
            jax.ShapeDtypeStruct((1, 1), jnp.float32),
        ],
    )(z, W)
    scale = 1.25 / (_B * _L * _D)
    return out, (loss[0, 0] * scale).astype(jnp.float32)
